# baseline (speedup 1.0000x reference)
import functools
import math

import jax
import jax.numpy as jnp
import numpy as np
from jax import lax
from jax.experimental import pallas as pl
from jax.experimental.pallas import tpu as pltpu
from jax.experimental.pallas import tpu_sc as plsc

_LS1 = [0, 1, 2]
_LS2 = [0, 1, 2]
_DIM1 = sum(2 * l + 1 for l in _LS1)
_DIM2 = sum(2 * l + 1 for l in _LS2)
_ODIM = _DIM1 * _DIM2


def _wigner3j(l1, l2, l3):
    f = math.factorial
    W = np.zeros((2 * l1 + 1, 2 * l2 + 1, 2 * l3 + 1), dtype=np.float64)
    pref = math.sqrt(
        f(l1 + l2 - l3) * f(l1 - l2 + l3) * f(-l1 + l2 + l3) / f(l1 + l2 + l3 + 1)
    )
    for m1 in range(-l1, l1 + 1):
        for m2 in range(-l2, l2 + 1):
            m3 = -(m1 + m2)
            if abs(m3) > l3:
                continue
            tmin = max(0, l2 - l3 - m1, l1 - l3 + m2)
            tmax = min(l1 + l2 - l3, l1 - m1, l2 + m2)
            s = 0.0
            for t in range(tmin, tmax + 1):
                s += ((-1) ** t) / (
                    f(t) * f(l3 - l2 + t + m1) * f(l3 - l1 + t - m2)
                    * f(l1 + l2 - l3 - t) * f(l1 - t - m1) * f(l2 - t + m2)
                )
            W[m1 + l1, m2 + l2, m3 + l3] = (
                ((-1) ** (l1 - l2 - m3)) * pref
                * math.sqrt(f(l1 + m1) * f(l1 - m1) * f(l2 + m2) * f(l2 - m2)
                            * f(l3 + m3) * f(l3 - m3)) * s
            )
    return W


def _build_columns():
    layout = {}
    i1off = 0
    for l1 in _LS1:
        i2off = 0
        for l2 in _LS2:
            for l3 in range(abs(l1 - l2), l1 + l2 + 1):
                layout.setdefault(l3, []).append((l1, l2, i1off, i2off))
            i2off += 2 * l2 + 1
        i1off += 2 * l1 + 1
    lmax2 = max(_LS2)
    cols = {}
    colkey = {}
    row_offset = 0
    for l3 in sorted(layout.keys()):
        for (l1, l2, o1, o2) in sorted(layout[l3], key=lambda x: x[0] * lmax2 + x[1]):
            C = _wigner3j(l1, l2, l3)
            scale = math.sqrt(2 * l3 + 1)
            for m3 in range(2 * l3 + 1):
                o = m3 + row_offset
                terms = []
                for m2 in range(2 * l2 + 1):
                    for m1 in range(2 * l1 + 1):
                        c = C[m1, m2, m3]
                        if c != 0.0:
                            terms.append((m1 + o1, m2 + o2, np.float32(c * scale)))
                if terms:
                    cols[o] = terms
                    colkey[o] = (l1, l2, l3, m3)
            row_offset += 2 * l3 + 1
    order = sorted(cols.keys(), key=lambda o: colkey[o])
    return [(o, cols[o]) for o in order]


_COLUMNS = _build_columns()

_NC = 2
_NS = 16
_NW = _NC * _NS
_LANES = 16
_CHUNK = 400
_SPLIT = 2


def _tile_body(n_rows, in1_hbm, in2_hbm, out_hbm, a1_v, a2_v, o_v):
    rows_per_w = n_rows // _NW
    n_chunks = rows_per_w // _CHUNK
    n_groups = _CHUNK // _LANES
    wid = lax.axis_index("s") * _NC + lax.axis_index("c")
    base = wid * rows_per_w
    lanes = lax.iota(jnp.int32, 16)
    lanes1 = lanes * _DIM1
    lanes2 = lanes * _DIM2
    laneso = lanes * _ODIM

    def group_body(g, _):
        r1 = lanes1 + g * (_LANES * _DIM1)
        r2 = lanes2 + g * (_LANES * _DIM2)
        ro = laneso + g * (_LANES * _ODIM)
        a1 = [plsc.load_gather(a1_v, [r1 + j]) for j in range(_DIM1)]
        a2 = [plsc.load_gather(a2_v, [r2 + j]) for j in range(_DIM2)]
        prod = {}
        for o, terms in _COLUMNS:
            acc = None
            for (i1, i2, c) in terms:
                p = prod.get((i1, i2))
                if p is None:
                    p = a1[i1] * a2[i2]
                    prod[(i1, i2)] = p
                t = p * c
                acc = t if acc is None else acc + t
            plsc.store_scatter(o_v, [ro + o], acc)
        return 0

    def chunk_body(ci, _):
        row0 = base + ci * _CHUNK
        pltpu.sync_copy(in1_hbm.at[pl.ds(row0 * _DIM1, _CHUNK * _DIM1)], a1_v)
        pltpu.sync_copy(in2_hbm.at[pl.ds(row0 * _DIM2, _CHUNK * _DIM2)], a2_v)
        lax.fori_loop(0, n_groups, group_body, 0)
        pltpu.sync_copy(o_v, out_hbm.at[pl.ds(row0 * _ODIM, _CHUNK * _ODIM)])
        return 0

    lax.fori_loop(0, n_chunks, chunk_body, 0)


def kernel(in1, in2):
    n = in1.shape[0]
    h = n // _SPLIT
    assert h % (_NW * _CHUNK) == 0
    run = pl.kernel(
        functools.partial(_tile_body, h),
        out_type=jax.ShapeDtypeStruct((h * _ODIM,), jnp.float32),
        mesh=plsc.VectorSubcoreMesh(
            core_axis_name="c", subcore_axis_name="s",
            num_cores=_NC, num_subcores=_NS,
        ),
        scratch_types=[
            pltpu.VMEM((_CHUNK * _DIM1,), jnp.float32),
            pltpu.VMEM((_CHUNK * _DIM2,), jnp.float32),
            pltpu.VMEM((_CHUNK * _ODIM,), jnp.float32),
        ],
        compiler_params=pltpu.CompilerParams(needs_layout_passes=False),
    )
    parts = []
    for s in range(_SPLIT):
        p1 = lax.slice_in_dim(in1, s * h, (s + 1) * h, axis=0)
        p2 = lax.slice_in_dim(in2, s * h, (s + 1) * h, axis=0)
        parts.append(run(p1.reshape(h * _DIM1), p2.reshape(h * _DIM2)))
    return jnp.concatenate(parts).reshape(n, _ODIM)


# speedup vs baseline: 1.1203x; 1.1203x over previous
import functools
import math

import jax
import jax.numpy as jnp
import numpy as np
from jax import lax
from jax.experimental import pallas as pl
from jax.experimental.pallas import tpu as pltpu
from jax.experimental.pallas import tpu_sc as plsc

_LS1 = [0, 1, 2]
_LS2 = [0, 1, 2]
_DIM1 = sum(2 * l + 1 for l in _LS1)
_DIM2 = sum(2 * l + 1 for l in _LS2)
_ODIM = _DIM1 * _DIM2


def _wigner3j(l1, l2, l3):
    f = math.factorial
    W = np.zeros((2 * l1 + 1, 2 * l2 + 1, 2 * l3 + 1), dtype=np.float64)
    pref = math.sqrt(
        f(l1 + l2 - l3) * f(l1 - l2 + l3) * f(-l1 + l2 + l3) / f(l1 + l2 + l3 + 1)
    )
    for m1 in range(-l1, l1 + 1):
        for m2 in range(-l2, l2 + 1):
            m3 = -(m1 + m2)
            if abs(m3) > l3:
                continue
            tmin = max(0, l2 - l3 - m1, l1 - l3 + m2)
            tmax = min(l1 + l2 - l3, l1 - m1, l2 + m2)
            s = 0.0
            for t in range(tmin, tmax + 1):
                s += ((-1) ** t) / (
                    f(t) * f(l3 - l2 + t + m1) * f(l3 - l1 + t - m2)
                    * f(l1 + l2 - l3 - t) * f(l1 - t - m1) * f(l2 - t + m2)
                )
            W[m1 + l1, m2 + l2, m3 + l3] = (
                ((-1) ** (l1 - l2 - m3)) * pref
                * math.sqrt(f(l1 + m1) * f(l1 - m1) * f(l2 + m2) * f(l2 - m2)
                            * f(l3 + m3) * f(l3 - m3)) * s
            )
    return W


def _build_columns():
    layout = {}
    i1off = 0
    for l1 in _LS1:
        i2off = 0
        for l2 in _LS2:
            for l3 in range(abs(l1 - l2), l1 + l2 + 1):
                layout.setdefault(l3, []).append((l1, l2, i1off, i2off))
            i2off += 2 * l2 + 1
        i1off += 2 * l1 + 1
    lmax2 = max(_LS2)
    cols = {}
    colkey = {}
    row_offset = 0
    for l3 in sorted(layout.keys()):
        for (l1, l2, o1, o2) in sorted(layout[l3], key=lambda x: x[0] * lmax2 + x[1]):
            C = _wigner3j(l1, l2, l3)
            scale = math.sqrt(2 * l3 + 1)
            for m3 in range(2 * l3 + 1):
                o = m3 + row_offset
                terms = []
                for m2 in range(2 * l2 + 1):
                    for m1 in range(2 * l1 + 1):
                        c = C[m1, m2, m3]
                        if c != 0.0:
                            terms.append((m1 + o1, m2 + o2, np.float32(c * scale)))
                if terms:
                    cols[o] = terms
                    colkey[o] = (l1, l2, l3, m3)
            row_offset += 2 * l3 + 1
    order = sorted(cols.keys(), key=lambda o: colkey[o])
    return [(o, cols[o]) for o in order]


_COLUMNS = _build_columns()

_NC = 2
_NS = 16
_NW = _NC * _NS
_LANES = 16
_CHUNK = 800


def _tile_body(n_rows, in1_hbm, in2_hbm, out_hbm, a1_v, a2_v, o_v):
    rows_per_w = n_rows // _NW
    n_chunks = rows_per_w // _CHUNK
    n_groups = _CHUNK // _LANES
    wid = lax.axis_index("s") * _NC + lax.axis_index("c")
    base = wid * rows_per_w
    lanes = lax.iota(jnp.int32, 16)
    lanes1 = lanes * _DIM1
    lanes2 = lanes * _DIM2
    laneso = lanes * _ODIM

    def group_body(g, _):
        r1 = lanes1 + g * (_LANES * _DIM1)
        r2 = lanes2 + g * (_LANES * _DIM2)
        ro = laneso + g * (_LANES * _ODIM)
        a1 = [plsc.load_gather(a1_v, [r1 + j]) for j in range(_DIM1)]
        a2 = [plsc.load_gather(a2_v, [r2 + j]) for j in range(_DIM2)]
        prod = {}
        for o, terms in _COLUMNS:
            acc = None
            for (i1, i2, c) in terms:
                p = prod.get((i1, i2))
                if p is None:
                    p = a1[i1] * a2[i2]
                    prod[(i1, i2)] = p
                t = p * c
                acc = t if acc is None else acc + t
            plsc.store_scatter(o_v, [ro + o], acc)
        return 0

    def chunk_body(ci, _):
        row0 = base + ci * _CHUNK
        pltpu.sync_copy(in1_hbm.at[pl.ds(row0 * _DIM1, _CHUNK * _DIM1)], a1_v)
        pltpu.sync_copy(in2_hbm.at[pl.ds(row0 * _DIM2, _CHUNK * _DIM2)], a2_v)
        lax.fori_loop(0, n_groups, group_body, 0)
        pltpu.sync_copy(o_v, out_hbm.at[pl.ds(row0 * _ODIM, _CHUNK * _ODIM)])
        return 0

    lax.fori_loop(0, n_chunks, chunk_body, 0)


def kernel(in1, in2):
    n = in1.shape[0]
    assert n % (_NW * _CHUNK) == 0
    run = pl.kernel(
        functools.partial(_tile_body, n),
        out_type=jax.ShapeDtypeStruct((n * _ODIM,), jnp.float32),
        mesh=plsc.VectorSubcoreMesh(
            core_axis_name="c", subcore_axis_name="s",
            num_cores=_NC, num_subcores=_NS,
        ),
        scratch_types=[
            pltpu.VMEM((_CHUNK * _DIM1,), jnp.float32),
            pltpu.VMEM((_CHUNK * _DIM2,), jnp.float32),
            pltpu.VMEM((_CHUNK * _ODIM,), jnp.float32),
        ],
        compiler_params=pltpu.CompilerParams(needs_layout_passes=False),
    )
    return run(in1.reshape(n * _DIM1), in2.reshape(n * _DIM2)).reshape(n, _ODIM)
